# baseline (device time: 143450 ns/iter reference)
import functools

import jax
import jax.numpy as jnp
from jax import lax
from jax.experimental import pallas as pl
from jax.experimental.pallas import tpu as pltpu

N_Z = 4
SCALE = 64 ** -0.5
WIRE_DTYPE = jnp.bfloat16

_CompilerParams = getattr(pltpu, "CompilerParams", None) or getattr(
    pltpu, "TPUCompilerParams"
)


def _pack(x):
    b, s, h, d = x.shape
    x = jnp.transpose(x, (0, 2, 1, 3))
    x = x.reshape(b, h // 2, 2, s, d)
    x = jnp.transpose(x, (0, 1, 3, 2, 4))
    return x.reshape(b, h // 2, s, 2 * d)


def kernel(Q, K, V):
    b, s_per, h, d = Q.shape
    hp = h // 2
    d2 = 2 * d

    Qp = _pack(Q * SCALE).astype(jnp.bfloat16)
    Kp = _pack(K).astype(jnp.bfloat16)
    Vp = _pack(V).astype(WIRE_DTYPE)

    def body(q_ref, k_ref, v_ref, out_ref, k_all, v_all,
             k_send, k_recv, v_send, v_recv):
        my_x = lax.axis_index("x")
        my_y = lax.axis_index("y")
        my_z = lax.axis_index("z")

        barrier = pltpu.get_barrier_semaphore()
        for off in range(1, N_Z):
            pl.semaphore_signal(
                barrier, inc=1,
                device_id=(my_x, my_y, (my_z + off) % N_Z),
                device_id_type=pl.DeviceIdType.MESH,
            )
        pl.semaphore_wait(barrier, N_Z - 1)

        k_all[my_z] = k_ref[...]
        v_all[my_z] = v_ref[...]

        rdmas = []
        for off in range(1, N_Z):
            tgt = (my_x, my_y, (my_z + off) % N_Z)
            for buf, ssem, rsem in (
                (k_all, k_send, k_recv),
                (v_all, v_send, v_recv),
            ):
                rdma = pltpu.make_async_remote_copy(
                    src_ref=buf.at[my_z],
                    dst_ref=buf.at[my_z],
                    send_sem=ssem.at[off - 1],
                    recv_sem=rsem.at[off - 1],
                    device_id=tgt,
                    device_id_type=pl.DeviceIdType.MESH,
                )
                rdma.start()
                rdmas.append(rdma)
        for rdma in rdmas:
            rdma.wait()

        def attn_block(i, carry):
            bb = i // hp
            pp = i % hp
            q2 = q_ref[bb, pp]
            k2 = jnp.concatenate(
                [k_all[zz, bb, pp] for zz in range(N_Z)], axis=0
            )
            v2 = jnp.concatenate(
                [v_all[zz, bb, pp] for zz in range(N_Z)], axis=0
            ).astype(jnp.bfloat16)
            halves = []
            for half in range(2):
                sl = slice(half * d, (half + 1) * d)
                s_mat = lax.dot_general(
                    q2[:, sl], k2[:, sl], (((1,), (1,)), ((), ())),
                    preferred_element_type=jnp.float32,
                )
                m = jnp.max(s_mat, axis=1, keepdims=True)
                p = jnp.exp(s_mat - m)
                denom = jnp.sum(p, axis=1, keepdims=True)
                o = lax.dot_general(
                    p.astype(jnp.bfloat16), v2[:, sl],
                    (((1,), (0,)), ((), ())),
                    preferred_element_type=jnp.float32,
                )
                halves.append(o / denom)
            out_ref[bb, pp] = jnp.concatenate(halves, axis=1)
            return carry

        lax.fori_loop(0, b * hp, attn_block, 0)

        @functools.partial(
            pl.run_scoped, second_barrier=pltpu.SemaphoreType.REGULAR
        )
        def _(second_barrier):
            for off in range(1, N_Z):
                pl.semaphore_signal(
                    second_barrier, inc=1,
                    device_id=(my_x, my_y, (my_z + off) % N_Z),
                    device_id_type=pl.DeviceIdType.MESH,
                )
            pl.semaphore_wait(second_barrier, N_Z - 1)

    out_p = pl.pallas_call(
        body,
        out_shape=jax.ShapeDtypeStruct((b, hp, s_per, d2), jnp.float32),
        in_specs=[pl.BlockSpec(memory_space=pltpu.VMEM)] * 3,
        out_specs=pl.BlockSpec(memory_space=pltpu.VMEM),
        scratch_shapes=[
            pltpu.VMEM((N_Z, b, hp, s_per, d2), jnp.bfloat16),
            pltpu.VMEM((N_Z, b, hp, s_per, d2), WIRE_DTYPE),
            pltpu.SemaphoreType.DMA((N_Z - 1,)),
            pltpu.SemaphoreType.DMA((N_Z - 1,)),
            pltpu.SemaphoreType.DMA((N_Z - 1,)),
            pltpu.SemaphoreType.DMA((N_Z - 1,)),
        ],
        compiler_params=_CompilerParams(
            collective_id=0, vmem_limit_bytes=100 * 1024 * 1024
        ),
    )(Qp, Kp, Vp)

    out_p = out_p.reshape(b, hp, s_per, 2, d)
    out_p = jnp.transpose(out_p, (0, 2, 1, 3, 4))
    return out_p.reshape(b, s_per, h, d)


# device time: 118711 ns/iter; 1.2084x vs baseline; 1.2084x over previous
import jax
import jax.numpy as jnp
from jax import lax
from jax.experimental import pallas as pl
from jax.experimental.pallas import tpu as pltpu

N_Z = 4
SCALE = 64 ** -0.5

_CompilerParams = getattr(pltpu, "CompilerParams", None) or getattr(
    pltpu, "TPUCompilerParams"
)


def _pack(x):
    b, s, h, d = x.shape
    x = jnp.transpose(x, (0, 2, 1, 3))
    x = x.reshape(b, h // 2, 2, s, d)
    x = jnp.transpose(x, (0, 1, 3, 2, 4))
    return x.reshape(b, h // 2, s, 2 * d)


def kernel(Q, K, V):
    b, s_per, h, d = Q.shape
    hp = h // 2
    d2 = 2 * d

    Qp = _pack(Q * SCALE).astype(jnp.bfloat16)
    Kp = _pack(K).astype(jnp.bfloat16)
    Vp = _pack(V).astype(jnp.bfloat16)

    def body(q_ref, k_ref, v_ref, out_ref, kv_all, send_sems, recv_sems):
        my_x = lax.axis_index("x")
        my_y = lax.axis_index("y")
        my_z = lax.axis_index("z")

        barrier = pltpu.get_barrier_semaphore()
        for off in range(1, N_Z):
            pl.semaphore_signal(
                barrier, inc=1,
                device_id=(my_x, my_y, (my_z + off) % N_Z),
                device_id_type=pl.DeviceIdType.MESH,
            )
        pl.semaphore_wait(barrier, N_Z - 1)

        kv_all[my_z, 0] = k_ref[...]
        kv_all[my_z, 1] = v_ref[...]

        rdmas = []
        for off in range(1, N_Z):
            rdma = pltpu.make_async_remote_copy(
                src_ref=kv_all.at[my_z],
                dst_ref=kv_all.at[my_z],
                send_sem=send_sems.at[off - 1],
                recv_sem=recv_sems.at[off - 1],
                device_id=(my_x, my_y, (my_z + off) % N_Z),
                device_id_type=pl.DeviceIdType.MESH,
            )
            rdma.start()
            rdmas.append(rdma)
        for rdma in rdmas:
            rdma.wait()

        def attn_block(i, carry):
            bb = i // hp
            pp = i % hp
            q2 = q_ref[bb, pp]
            k2 = jnp.concatenate(
                [kv_all[zz, 0, bb, pp] for zz in range(N_Z)], axis=0
            )
            v2 = jnp.concatenate(
                [kv_all[zz, 1, bb, pp] for zz in range(N_Z)], axis=0
            )
            halves = []
            for half in range(2):
                sl = slice(half * d, (half + 1) * d)
                s_mat = lax.dot_general(
                    q2[:, sl], k2[:, sl], (((1,), (1,)), ((), ())),
                    preferred_element_type=jnp.float32,
                )
                m = jnp.max(s_mat, axis=1, keepdims=True)
                p = jnp.exp(s_mat - m)
                denom = jnp.sum(p, axis=1, keepdims=True)
                o = lax.dot_general(
                    p.astype(jnp.bfloat16), v2[:, sl],
                    (((1,), (0,)), ((), ())),
                    preferred_element_type=jnp.float32,
                )
                halves.append(o / denom)
            out_ref[bb, pp] = jnp.concatenate(halves, axis=1)
            return carry

        lax.fori_loop(0, b * hp, attn_block, 0)

    out_p = pl.pallas_call(
        body,
        out_shape=jax.ShapeDtypeStruct((b, hp, s_per, d2), jnp.float32),
        in_specs=[pl.BlockSpec(memory_space=pltpu.VMEM)] * 3,
        out_specs=pl.BlockSpec(memory_space=pltpu.VMEM),
        scratch_shapes=[
            pltpu.VMEM((N_Z, 2, b, hp, s_per, d2), jnp.bfloat16),
            pltpu.SemaphoreType.DMA((N_Z - 1,)),
            pltpu.SemaphoreType.DMA((N_Z - 1,)),
        ],
        compiler_params=_CompilerParams(
            collective_id=0, vmem_limit_bytes=100 * 1024 * 1024
        ),
    )(Qp, Kp, Vp)

    out_p = out_p.reshape(b, hp, s_per, 2, d)
    out_p = jnp.transpose(out_p, (0, 2, 1, 3, 4))
    return out_p.reshape(b, s_per, h, d)
